# baseline (speedup 1.0000x reference)
import functools

import jax
import jax.numpy as jnp
from jax.experimental import pallas as pl
from jax.experimental.pallas import tpu as pltpu

B, S, D = 1, 2048, 1024
E, K = 8, 2
F = 512
NE = E + 2
TM = 256
T = S // TM


def _top2(logits):
    iota = jax.lax.broadcasted_iota(jnp.int32, logits.shape, 1)
    m1 = jnp.max(logits, axis=-1, keepdims=True)
    i1 = jnp.min(jnp.where(logits == m1, iota, E), axis=-1, keepdims=True)
    l2 = jnp.where(iota == i1, -jnp.inf, logits)
    m2 = jnp.max(l2, axis=-1, keepdims=True)
    i2 = jnp.min(jnp.where(l2 == m2, iota, E), axis=-1, keepdims=True)
    w1 = 1.0 / (1.0 + jnp.exp(m2 - m1))
    w2 = 1.0 - w1
    return i1, i2, w1, w2


def _moe_body(x_ref, wr_ref, wg_ref, wu_ref, wd_ref, wsg_ref,
              logits_ref, out_ref):
    e_id = pl.program_id(0)
    t_id = pl.program_id(1)
    x = x_ref[...]
    logits = jnp.dot(x, wr_ref[...], preferred_element_type=jnp.float32)
    logits_ref[...] = logits
    i1, i2, w1, w2 = _top2(logits)
    combine = jnp.sum(
        jnp.where(i1 == e_id, w1, 0.0) + jnp.where(i2 == e_id, w2, 0.0),
        axis=-1, keepdims=True)
    gate = jax.nn.sigmoid(jnp.dot(x, wsg_ref[...],
                                  preferred_element_type=jnp.float32))
    w_col = jnp.where(e_id < E, combine, gate)
    xb = x.astype(jnp.bfloat16)
    g = jnp.dot(xb, wg_ref[0].astype(jnp.bfloat16),
                preferred_element_type=jnp.float32)
    u = jnp.dot(xb, wu_ref[0].astype(jnp.bfloat16),
                preferred_element_type=jnp.float32)
    h = jax.nn.silu(g) * u
    o = jnp.dot(h.astype(jnp.bfloat16), wd_ref[0].astype(jnp.bfloat16),
                preferred_element_type=jnp.float32)
    contrib = w_col * o
    rows = pl.ds(t_id * TM, TM)

    @pl.when(e_id == 0)
    def _init():
        out_ref[rows, :] = contrib

    @pl.when(e_id != 0)
    def _acc():
        out_ref[rows, :] = out_ref[rows, :] + contrib


def _dense_moe(x, W_router, WgA, WuA, WdA, Wse_gate):
    return pl.pallas_call(
        _moe_body,
        grid=(NE, T),
        in_specs=[
            pl.BlockSpec((TM, D), lambda e, t: (t, 0)),
            pl.BlockSpec((D, E), lambda e, t: (0, 0)),
            pl.BlockSpec((1, D, F), lambda e, t: (e, 0, 0)),
            pl.BlockSpec((1, D, F), lambda e, t: (e, 0, 0)),
            pl.BlockSpec((1, F, D), lambda e, t: (e, 0, 0)),
            pl.BlockSpec((D, 1), lambda e, t: (0, 0)),
        ],
        out_specs=[
            pl.BlockSpec((TM, E), lambda e, t: (t, 0)),
            pl.BlockSpec((S, D), lambda e, t: (0, 0)),
        ],
        out_shape=[
            jax.ShapeDtypeStruct((S, E), jnp.float32),
            jax.ShapeDtypeStruct((S, D), jnp.float32),
        ],
        compiler_params=pltpu.CompilerParams(
            dimension_semantics=("arbitrary", "arbitrary")),
    )(x, W_router, WgA, WuA, WdA, Wse_gate)


@jax.jit
def kernel(hidden_states, W_router, Wg, Wu, Wd, Wsg, Wsu, Wsd, Wse_gate):
    x = hidden_states.reshape(S, D)
    WgA = jnp.concatenate(
        [Wg, Wsg[:, :F][None], Wsg[:, F:][None]], axis=0)
    WuA = jnp.concatenate(
        [Wu, Wsu[:, :F][None], Wsu[:, F:][None]], axis=0)
    WdA = jnp.concatenate(
        [Wd, Wsd[:F, :][None], Wsd[F:, :][None]], axis=0)
    logits, out = _dense_moe(x, W_router, WgA, WuA, WdA, Wse_gate)
    return out.reshape(B, S, D), logits.reshape(B, S, E)


# speedup vs baseline: 1.3428x; 1.3428x over previous
import jax
import jax.numpy as jnp
from jax.experimental import pallas as pl
from jax.experimental.pallas import tpu as pltpu

B, S, D = 1, 2048, 1024
E, K = 8, 2
F = 512
F_SH = 1024
TM = 256
T = S // TM


def _top2(logits):
    iota = jax.lax.broadcasted_iota(jnp.int32, logits.shape, 1)
    m1 = jnp.max(logits, axis=-1, keepdims=True)
    i1 = jnp.min(jnp.where(logits == m1, iota, E), axis=-1, keepdims=True)
    l2 = jnp.where(iota == i1, -jnp.inf, logits)
    m2 = jnp.max(l2, axis=-1, keepdims=True)
    i2 = jnp.min(jnp.where(l2 == m2, iota, E), axis=-1, keepdims=True)
    w1 = 1.0 / (1.0 + jnp.exp(m2 - m1))
    w2 = 1.0 - w1
    return i1, i2, w1, w2


def _moe_body(x_ref, wr_ref, wsgate_ref, wg_ref, wu_ref, wd_ref,
              wsg_ref, wsu_ref, wsd_ref, logits_ref, out_ref, wtab_ref):
    e_id = pl.program_id(0)
    t_id = pl.program_id(1)
    rows = pl.ds(t_id * TM, TM)
    x = x_ref[...]

    @pl.when(e_id == 0)
    def _router():
        logits = jnp.dot(x, wr_ref[...], preferred_element_type=jnp.float32)
        logits_ref[...] = logits
        i1, i2, w1, w2 = _top2(logits)
        gate = jax.nn.sigmoid(jnp.dot(x, wsgate_ref[...],
                                      preferred_element_type=jnp.float32))
        cols = jax.lax.broadcasted_iota(jnp.int32, (TM, 16), 1)
        wtab = (jnp.where(cols == i1, w1, 0.0)
                + jnp.where(cols == i2, w2, 0.0)
                + jnp.where(cols == E, gate, 0.0))
        wtab_ref[rows, :] = wtab

    cols16 = jax.lax.broadcasted_iota(jnp.int32, (TM, 16), 1)
    w_col = jnp.sum(jnp.where(cols16 == e_id, wtab_ref[rows, :], 0.0),
                    axis=-1, keepdims=True)

    @pl.when(e_id < E)
    def _expert():
        g = jnp.dot(x, wg_ref[0], preferred_element_type=jnp.float32)
        u = jnp.dot(x, wu_ref[0], preferred_element_type=jnp.float32)
        h = jax.nn.silu(g) * u
        o = jnp.dot(h, wd_ref[0], preferred_element_type=jnp.float32)
        contrib = w_col * o

        @pl.when(e_id == 0)
        def _init():
            out_ref[rows, :] = contrib

        @pl.when(e_id != 0)
        def _acc():
            out_ref[rows, :] = out_ref[rows, :] + contrib

    @pl.when(e_id == E)
    def _shared():
        g = jnp.dot(x, wsg_ref[...], preferred_element_type=jnp.float32)
        u = jnp.dot(x, wsu_ref[...], preferred_element_type=jnp.float32)
        h = jax.nn.silu(g) * u
        o = jnp.dot(h, wsd_ref[...], preferred_element_type=jnp.float32)
        out_ref[rows, :] = out_ref[rows, :] + w_col * o


def _dense_moe(x, W_router, Wse_gate, Wg, Wu, Wd, Wsg, Wsu, Wsd):
    emap = lambda e, t: (jnp.minimum(e, E - 1), 0, 0)
    return pl.pallas_call(
        _moe_body,
        grid=(E + 1, T),
        in_specs=[
            pl.BlockSpec((TM, D), lambda e, t: (t, 0)),
            pl.BlockSpec((D, E), lambda e, t: (0, 0)),
            pl.BlockSpec((D, 1), lambda e, t: (0, 0)),
            pl.BlockSpec((1, D, F), emap),
            pl.BlockSpec((1, D, F), emap),
            pl.BlockSpec((1, F, D), emap),
            pl.BlockSpec((D, F_SH), lambda e, t: (0, 0)),
            pl.BlockSpec((D, F_SH), lambda e, t: (0, 0)),
            pl.BlockSpec((F_SH, D), lambda e, t: (0, 0)),
        ],
        out_specs=[
            pl.BlockSpec((TM, E), lambda e, t: (t, 0)),
            pl.BlockSpec((S, D), lambda e, t: (0, 0)),
        ],
        out_shape=[
            jax.ShapeDtypeStruct((S, E), jnp.float32),
            jax.ShapeDtypeStruct((S, D), jnp.float32),
        ],
        scratch_shapes=[pltpu.VMEM((S, 16), jnp.float32)],
        compiler_params=pltpu.CompilerParams(
            dimension_semantics=("arbitrary", "arbitrary")),
    )(x, W_router, Wse_gate, Wg, Wu, Wd, Wsg, Wsu, Wsd)


@jax.jit
def kernel(hidden_states, W_router, Wg, Wu, Wd, Wsg, Wsu, Wsd, Wse_gate):
    x = hidden_states.reshape(S, D)
    logits, out = _dense_moe(x, W_router, Wse_gate, Wg, Wu, Wd, Wsg, Wsu, Wsd)
    return out.reshape(B, S, D), logits.reshape(B, S, E)
